# Initial kernel scaffold; baseline (speedup 1.0000x reference)
import jax
import jax.numpy as jnp
from jax.experimental import pallas as pl


def kernel(x, edge_index, W1, b1, W2, b2, Wc1, bc1, Wc2, bc2):
    raise NotImplementedError("write your pallas kernel here")


# speedup vs baseline: 10.7425x; 10.7425x over previous
import functools

import jax
import jax.numpy as jnp
from jax import lax
from jax.experimental import pallas as pl
from jax.experimental.pallas import tpu as pltpu
from jax.experimental.pallas import tpu_sc as plsc

N = 10000
E = 320000
EB = 128
NBLK = E // EB
NS = 16
NC = 2
RPT = 624
RPT_LAST = N - 15 * RPT
D_IN = 128
D_HID = 256
DH = D_HID // 2


@functools.cache
def _build_deg_kernel():
    @functools.partial(
        pl.kernel,
        mesh=plsc.VectorSubcoreMesh(core_axis_name="c", subcore_axis_name="s"),
        out_type=jax.ShapeDtypeStruct((NC * N, DH), jnp.float32),
        scratch_types=[
            pltpu.VMEM((EB,), jnp.int32),
            pltpu.VMEM((EB, DH), jnp.float32),
            pltpu.VMEM_SHARED((N, DH), jnp.float32),
        ],
    )
    def deg_kernel(dst_hbm, ones_hbm, zeros_hbm, out_hbm, idx_v, ones_v, acc):
        c = lax.axis_index("c")
        s = lax.axis_index("s")
        r0 = pl.multiple_of(s * RPT, 8)

        def stripe_copy(mk_src, mk_dst):
            @pl.when(s < NS - 1)
            def _():
                pltpu.sync_copy(mk_src(RPT), mk_dst(RPT))

            @pl.when(s == NS - 1)
            def _():
                pltpu.sync_copy(mk_src(RPT_LAST), mk_dst(RPT_LAST))

        stripe_copy(lambda nr: zeros_hbm.at[pl.ds(r0, nr)],
                    lambda nr: acc.at[pl.ds(r0, nr)])
        pltpu.sync_copy(ones_hbm, ones_v)
        plsc.subcore_barrier()

        half = NBLK // NC

        def body(i, carry):
            off = s + i * NS

            @pl.when(off < half)
            def _():
                b = c * half + off
                pltpu.sync_copy(dst_hbm.at[pl.ds(b * EB, EB)], idx_v)
                pltpu.sync_copy(ones_v, acc.at[idx_v], add=True)

            return carry

        lax.fori_loop(0, (half + NS - 1) // NS, body, 0)
        plsc.subcore_barrier()
        stripe_copy(lambda nr: acc.at[pl.ds(r0, nr)],
                    lambda nr: out_hbm.at[pl.ds(c * N + r0, nr)])

    return deg_kernel


@functools.cache
def _build_agg_kernel():
    @functools.partial(
        pl.kernel,
        mesh=plsc.VectorSubcoreMesh(core_axis_name="c", subcore_axis_name="s"),
        out_type=(
            jax.ShapeDtypeStruct((N, DH), jnp.float32),
            jax.ShapeDtypeStruct((N, DH), jnp.float32),
        ),
        scratch_types=[
            pltpu.VMEM((EB, DH), jnp.float32),
            pltpu.VMEM((EB,), jnp.int32),
            pltpu.VMEM((EB,), jnp.int32),
            pltpu.VMEM_SHARED((N, DH), jnp.float32),
            pltpu.SemaphoreType.DMA,
        ],
    )
    def agg_kernel(g0_hbm, g1_hbm, src_hbm, dst_hbm, o0_hbm, o1_hbm,
                   rows_v, sidx_v, didx_v, acc, sem):
        c = lax.axis_index("c")
        s = lax.axis_index("s")
        r0 = pl.multiple_of(s * RPT, 8)

        def stripe_copy(mk_src, mk_dst):
            @pl.when(s < NS - 1)
            def _():
                pltpu.sync_copy(mk_src(RPT), mk_dst(RPT))

            @pl.when(s == NS - 1)
            def _():
                pltpu.sync_copy(mk_src(RPT_LAST), mk_dst(RPT_LAST))

        def run(g_hbm, o_hbm):
            stripe_copy(lambda nr: g_hbm.at[pl.ds(r0, nr)],
                        lambda nr: acc.at[pl.ds(r0, nr)])
            plsc.subcore_barrier()

            def body(i, carry):
                b = s + i * NS

                @pl.when(b < NBLK)
                def _():
                    pltpu.sync_copy(src_hbm.at[pl.ds(b * EB, EB)], sidx_v)
                    pltpu.sync_copy(dst_hbm.at[pl.ds(b * EB, EB)], didx_v)
                    pltpu.async_copy(g_hbm.at[sidx_v], rows_v, sem).wait()
                    pltpu.sync_copy(rows_v, acc.at[didx_v], add=True)

                return carry

            lax.fori_loop(0, (NBLK + NS - 1) // NS, body, 0)
            plsc.subcore_barrier()
            stripe_copy(lambda nr: acc.at[pl.ds(r0, nr)],
                        lambda nr: o_hbm.at[pl.ds(r0, nr)])

        @pl.when(c == 0)
        def _():
            run(g0_hbm, o0_hbm)

        @pl.when(c == 1)
        def _():
            run(g1_hbm, o1_hbm)

    return agg_kernel


_RB = 1000
_GRID = N // _RB


def _tca_body(deg0, deg1, x, w1, d_out, g0_out, g1_out):
    deg = 1.0 + deg0[:, 0] + deg1[:, 0]
    dv = lax.rsqrt(deg)[:, None]
    h = jnp.dot(x[...], w1[...], preferred_element_type=jnp.float32)
    g = h * dv
    d_out[...] = dv
    g0_out[...] = g[:, :DH]
    g1_out[...] = g[:, DH:]


def _tcb_body(s0, s1, d, b1, w2, g0_out, g1_out):
    dv = d[...]
    a = jnp.concatenate([s0[...], s1[...]], axis=1) * dv + b1[...]
    a = jnp.maximum(a, 0.0)
    h = jnp.dot(a, w2[...], preferred_element_type=jnp.float32)
    g = h * dv
    g0_out[...] = g[:, :DH]
    g1_out[...] = g[:, DH:]


def _tcc_body(s0, s1, d, b2, wc1, bc1, wc2, bc2, out):
    dv = d[...]
    a = jnp.concatenate([s0[...], s1[...]], axis=1) * dv + b2[...]
    a = jnp.maximum(a, 0.0)
    h = jnp.dot(a, wc1[...], preferred_element_type=jnp.float32) + bc1[...]
    h = 0.5 * h * (1.0 + lax.erf(h * (2.0 ** -0.5)))
    o = jnp.dot(h, wc2[...], preferred_element_type=jnp.float32) + bc2[...]
    out[...] = jax.nn.sigmoid(o)


def _row_spec(cols):
    return pl.BlockSpec((_RB, cols), lambda i: (i, 0))


def _full_spec(r, cols):
    return pl.BlockSpec((r, cols), lambda i: (0, 0))


_tca_call = pl.pallas_call(
    _tca_body,
    grid=(_GRID,),
    in_specs=[_row_spec(DH), _row_spec(DH), _row_spec(D_IN), _full_spec(D_IN, D_HID)],
    out_specs=[_row_spec(1), _row_spec(DH), _row_spec(DH)],
    out_shape=[
        jax.ShapeDtypeStruct((N, 1), jnp.float32),
        jax.ShapeDtypeStruct((N, DH), jnp.float32),
        jax.ShapeDtypeStruct((N, DH), jnp.float32),
    ],
)

_tcb_call = pl.pallas_call(
    _tcb_body,
    grid=(_GRID,),
    in_specs=[_row_spec(DH), _row_spec(DH), _row_spec(1), _full_spec(1, D_HID),
              _full_spec(D_HID, D_HID)],
    out_specs=[_row_spec(DH), _row_spec(DH)],
    out_shape=[
        jax.ShapeDtypeStruct((N, DH), jnp.float32),
        jax.ShapeDtypeStruct((N, DH), jnp.float32),
    ],
)

_tcc_call = pl.pallas_call(
    _tcc_body,
    grid=(_GRID,),
    in_specs=[_row_spec(DH), _row_spec(DH), _row_spec(1), _full_spec(1, D_HID),
              _full_spec(D_HID, 64), _full_spec(1, 64), _full_spec(64, 1),
              _full_spec(1, 1)],
    out_specs=[_row_spec(1)],
    out_shape=[jax.ShapeDtypeStruct((N, 1), jnp.float32)],
)


def kernel(x, edge_index, W1, b1, W2, b2, Wc1, bc1, Wc2, bc2):
    edge_index = edge_index.astype(jnp.int32)
    src = edge_index[0]
    dst = edge_index[1]
    ones = jnp.ones((EB, DH), jnp.float32)
    zeros = jnp.zeros((N, DH), jnp.float32)

    degp = _build_deg_kernel()(dst, ones, zeros)
    d, g0, g1 = _tca_call(degp[:N], degp[N:], x, W1)
    s0, s1 = _build_agg_kernel()(g0, g1, src, dst)
    t0, t1 = _tcb_call(s0, s1, d, b1.reshape(1, D_HID), W2)
    u0, u1 = _build_agg_kernel()(t0, t1, src, dst)
    (out,) = _tcc_call(u0, u1, d, b2.reshape(1, D_HID), Wc1, bc1.reshape(1, 64),
                       Wc2, bc2.reshape(1, 1))
    return out


# speedup vs baseline: 13.7071x; 1.2760x over previous
import functools

import jax
import jax.numpy as jnp
from jax import lax
from jax.experimental import pallas as pl
from jax.experimental.pallas import tpu as pltpu
from jax.experimental.pallas import tpu_sc as plsc

N = 10000
E = 320000
EB = 128
NBLK = E // EB
NS = 16
NC = 2
RPT = 624
RPT_LAST = N - 15 * RPT
D_IN = 128
D_HID = 256
DH = D_HID // 2


@functools.cache
def _build_deg_kernel():
    @functools.partial(
        pl.kernel,
        mesh=plsc.VectorSubcoreMesh(core_axis_name="c", subcore_axis_name="s"),
        out_type=jax.ShapeDtypeStruct((NC * N, DH), jnp.float32),
        scratch_types=[
            pltpu.VMEM((EB,), jnp.int32),
            pltpu.VMEM((EB, DH), jnp.float32),
            pltpu.VMEM_SHARED((N, DH), jnp.float32),
        ],
    )
    def deg_kernel(dst_hbm, ones_hbm, zeros_hbm, out_hbm, idx_v, ones_v, acc):
        c = lax.axis_index("c")
        s = lax.axis_index("s")
        r0 = pl.multiple_of(s * RPT, 8)

        def stripe_copy(mk_src, mk_dst):
            @pl.when(s < NS - 1)
            def _():
                pltpu.sync_copy(mk_src(RPT), mk_dst(RPT))

            @pl.when(s == NS - 1)
            def _():
                pltpu.sync_copy(mk_src(RPT_LAST), mk_dst(RPT_LAST))

        stripe_copy(lambda nr: zeros_hbm.at[pl.ds(r0, nr)],
                    lambda nr: acc.at[pl.ds(r0, nr)])
        pltpu.sync_copy(ones_hbm, ones_v)
        plsc.subcore_barrier()

        half = NBLK // NC

        def body(i, carry):
            off = s + i * NS

            @pl.when(off < half)
            def _():
                b = c * half + off
                pltpu.sync_copy(dst_hbm.at[pl.ds(b * EB, EB)], idx_v)
                pltpu.sync_copy(ones_v, acc.at[idx_v], add=True)

            return carry

        lax.fori_loop(0, (half + NS - 1) // NS, body, 0)
        plsc.subcore_barrier()
        stripe_copy(lambda nr: acc.at[pl.ds(r0, nr)],
                    lambda nr: out_hbm.at[pl.ds(c * N + r0, nr)])

    return deg_kernel


IG = 16
BPT = 160
NGRP = BPT // IG
NBLK_PAD = NS * BPT // 16 * 16 + IG * 2


@functools.cache
def _build_agg_kernel():
    @functools.partial(
        pl.kernel,
        mesh=plsc.VectorSubcoreMesh(core_axis_name="c", subcore_axis_name="s"),
        out_type=(
            jax.ShapeDtypeStruct((N, DH), jnp.float32),
            jax.ShapeDtypeStruct((N, DH), jnp.float32),
        ),
        scratch_types=[
            pltpu.VMEM((EB, DH), jnp.float32),
            pltpu.VMEM((IG, EB), jnp.int32),
            pltpu.VMEM((IG, EB), jnp.int32),
            pltpu.VMEM_SHARED((N, DH), jnp.float32),
            pltpu.SemaphoreType.DMA,
        ],
    )
    def agg_kernel(g0_hbm, g1_hbm, src2_hbm, dst2_hbm, o0_hbm, o1_hbm,
                   rows0, sidx_v, didx_v, acc, gsem0):
        c = lax.axis_index("c")
        s = lax.axis_index("s")
        r0 = pl.multiple_of(s * RPT, 8)

        def stripe_copy(mk_src, mk_dst):
            @pl.when(s < NS - 1)
            def _():
                pltpu.sync_copy(mk_src(RPT), mk_dst(RPT))

            @pl.when(s == NS - 1)
            def _():
                pltpu.sync_copy(mk_src(RPT_LAST), mk_dst(RPT_LAST))

        def run(g_hbm, o_hbm):
            stripe_copy(lambda nr: g_hbm.at[pl.ds(r0, nr)],
                        lambda nr: acc.at[pl.ds(r0, nr)])
            plsc.subcore_barrier()

            b0 = s * BPT

            def group(g, carry):
                gb = pl.multiple_of(b0 + g * IG, 8)

                @pl.when(gb < NBLK)
                def _():
                    pltpu.sync_copy(src2_hbm.at[pl.ds(gb, IG)], sidx_v)
                    pltpu.sync_copy(dst2_hbm.at[pl.ds(gb, IG)], didx_v)

                    for j in range(IG):
                        @pl.when(gb + j < NBLK)
                        def _():
                            pltpu.async_copy(
                                g_hbm.at[sidx_v.at[j]], rows0, gsem0).wait()
                            pltpu.sync_copy(rows0, acc.at[didx_v.at[j]],
                                            add=True)

                return carry

            lax.fori_loop(0, NGRP, group, 0)
            plsc.subcore_barrier()
            stripe_copy(lambda nr: acc.at[pl.ds(r0, nr)],
                        lambda nr: o_hbm.at[pl.ds(r0, nr)])

        @pl.when(c == 0)
        def _():
            run(g0_hbm, o0_hbm)

        @pl.when(c == 1)
        def _():
            run(g1_hbm, o1_hbm)

    return agg_kernel


_RB = 1000
_GRID = N // _RB


def _tca_body(deg0, deg1, x, w1, d_out, g0_out, g1_out):
    deg = 1.0 + deg0[:, 0] + deg1[:, 0]
    dv = lax.rsqrt(deg)[:, None]
    h = jnp.dot(x[...], w1[...], preferred_element_type=jnp.float32)
    g = h * dv
    d_out[...] = dv
    g0_out[...] = g[:, :DH]
    g1_out[...] = g[:, DH:]


def _tcb_body(s0, s1, d, b1, w2, g0_out, g1_out):
    dv = d[...]
    a = jnp.concatenate([s0[...], s1[...]], axis=1) * dv + b1[...]
    a = jnp.maximum(a, 0.0)
    h = jnp.dot(a, w2[...], preferred_element_type=jnp.float32)
    g = h * dv
    g0_out[...] = g[:, :DH]
    g1_out[...] = g[:, DH:]


def _tcc_body(s0, s1, d, b2, wc1, bc1, wc2, bc2, out):
    dv = d[...]
    a = jnp.concatenate([s0[...], s1[...]], axis=1) * dv + b2[...]
    a = jnp.maximum(a, 0.0)
    h = jnp.dot(a, wc1[...], preferred_element_type=jnp.float32) + bc1[...]
    h = 0.5 * h * (1.0 + lax.erf(h * (2.0 ** -0.5)))
    o = jnp.dot(h, wc2[...], preferred_element_type=jnp.float32) + bc2[...]
    out[...] = jax.nn.sigmoid(o)


def _row_spec(cols):
    return pl.BlockSpec((_RB, cols), lambda i: (i, 0))


def _full_spec(r, cols):
    return pl.BlockSpec((r, cols), lambda i: (0, 0))


_tca_call = pl.pallas_call(
    _tca_body,
    grid=(_GRID,),
    in_specs=[_row_spec(DH), _row_spec(DH), _row_spec(D_IN), _full_spec(D_IN, D_HID)],
    out_specs=[_row_spec(1), _row_spec(DH), _row_spec(DH)],
    out_shape=[
        jax.ShapeDtypeStruct((N, 1), jnp.float32),
        jax.ShapeDtypeStruct((N, DH), jnp.float32),
        jax.ShapeDtypeStruct((N, DH), jnp.float32),
    ],
)

_tcb_call = pl.pallas_call(
    _tcb_body,
    grid=(_GRID,),
    in_specs=[_row_spec(DH), _row_spec(DH), _row_spec(1), _full_spec(1, D_HID),
              _full_spec(D_HID, D_HID)],
    out_specs=[_row_spec(DH), _row_spec(DH)],
    out_shape=[
        jax.ShapeDtypeStruct((N, DH), jnp.float32),
        jax.ShapeDtypeStruct((N, DH), jnp.float32),
    ],
)

_tcc_call = pl.pallas_call(
    _tcc_body,
    grid=(_GRID,),
    in_specs=[_row_spec(DH), _row_spec(DH), _row_spec(1), _full_spec(1, D_HID),
              _full_spec(D_HID, 64), _full_spec(1, 64), _full_spec(64, 1),
              _full_spec(1, 1)],
    out_specs=[_row_spec(1)],
    out_shape=[jax.ShapeDtypeStruct((N, 1), jnp.float32)],
)


def kernel(x, edge_index, W1, b1, W2, b2, Wc1, bc1, Wc2, bc2):
    edge_index = edge_index.astype(jnp.int32)
    src = edge_index[0]
    dst = edge_index[1]
    ones = jnp.ones((EB, DH), jnp.float32)
    zeros = jnp.zeros((N, DH), jnp.float32)

    pad = ((0, NBLK_PAD - NBLK), (0, 0))
    src2 = jnp.pad(src.reshape(NBLK, EB), pad)
    dst2 = jnp.pad(dst.reshape(NBLK, EB), pad)

    degp = _build_deg_kernel()(dst, ones, zeros)
    d, g0, g1 = _tca_call(degp[:N], degp[N:], x, W1)
    s0, s1 = _build_agg_kernel()(g0, g1, src2, dst2)
    t0, t1 = _tcb_call(s0, s1, d, b1.reshape(1, D_HID), W2)
    u0, u1 = _build_agg_kernel()(t0, t1, src2, dst2)
    (out,) = _tcc_call(u0, u1, d, b2.reshape(1, D_HID), Wc1, bc1.reshape(1, 64),
                       Wc2, bc2.reshape(1, 1))
    return out
